# baseline (speedup 1.0000x reference)
import functools

import jax
import jax.numpy as jnp
from jax import lax
from jax.experimental import pallas as pl
from jax.experimental.pallas import tpu as pltpu
from jax.experimental.pallas import tpu_sc as plsc

B = 16384
NF = 26
FIELD_SIZE = 100000
NC = 2
NS = 16
NW = NC * NS
ROWS_W = B // NW
FLAT_W = ROWS_W * NF
CH = 128
NCH = FLAT_W // CH
LANES = 16


def _emb_sum_sc(x3, tab):
    mesh = plsc.VectorSubcoreMesh(
        core_axis_name="c", subcore_axis_name="s", num_cores=NC, num_subcores=NS
    )

    @functools.partial(
        pl.kernel,
        out_type=jax.ShapeDtypeStruct((B,), jnp.float32),
        mesh=mesh,
        scratch_types=[
            pltpu.VMEM((NCH, CH), jnp.int32),
            pltpu.VMEM((FLAT_W,), jnp.float32),
            pltpu.VMEM((ROWS_W,), jnp.float32),
            pltpu.SemaphoreType.DMA,
        ],
    )
    def k(x_hbm, tab_hbm, out_hbm, idx_v, rows_v, acc_v, sem):
        wid = lax.axis_index("s") * NC + lax.axis_index("c")
        base = wid * ROWS_W
        pltpu.sync_copy(x_hbm.at[wid], idx_v)

        def gather_chunk(j, carry):
            off = (j // (ROWS_W // CH)) * FIELD_SIZE
            for c in range(CH // LANES):
                idx_v[j, pl.ds(c * LANES, LANES)] = (
                    idx_v[j, pl.ds(c * LANES, LANES)] + off
                )
            pltpu.async_copy(
                tab_hbm.at[idx_v.at[j]], rows_v.at[pl.ds(j * CH, CH)], sem
            ).wait()
            return carry

        lax.fori_loop(0, NCH, gather_chunk, 0)

        def row_sum(g, carry):
            acc = rows_v[pl.ds(g * LANES, LANES)]
            for f in range(1, NF):
                acc = acc + rows_v[pl.ds(f * ROWS_W + g * LANES, LANES)]
            acc_v[pl.ds(g * LANES, LANES)] = acc
            return carry

        lax.fori_loop(0, ROWS_W // LANES, row_sum, 0)
        pltpu.sync_copy(acc_v, out_hbm.at[pl.ds(base, ROWS_W)])

    return k(x3, tab)


def _tc_body(xc_ref, w_ref, b_ref, emb_ref, o_ref):
    s = jnp.sum(xc_ref[...] * w_ref[...], axis=1, keepdims=True)
    o_ref[...] = s + emb_ref[...] + b_ref[0, 0]


def _linear_tc(x_cont, w2, bias2, emb2):
    blk = 1024
    return pl.pallas_call(
        _tc_body,
        grid=(B // blk,),
        in_specs=[
            pl.BlockSpec((blk, 128), lambda i: (i, 0)),
            pl.BlockSpec((1, 128), lambda i: (0, 0)),
            pl.BlockSpec((1, 1), lambda i: (0, 0)),
            pl.BlockSpec((blk, 1), lambda i: (i, 0)),
        ],
        out_specs=pl.BlockSpec((blk, 1), lambda i: (i, 0)),
        out_shape=jax.ShapeDtypeStruct((B, 1), jnp.float32),
    )(x_cont, w2, bias2, emb2)


def kernel(x, x_cont, emb_x, table, w, bias):
    x3 = x.reshape(NW, ROWS_W, NF).transpose(0, 2, 1).reshape(NW, NCH, CH)
    tab = table.reshape(-1)
    emb = _emb_sum_sc(x3, tab)
    return _linear_tc(x_cont, w.reshape(1, 128), bias.reshape(1, 1), emb.reshape(B, 1))


# speedup vs baseline: 1.1888x; 1.1888x over previous
import functools

import numpy as np

import jax
import jax.numpy as jnp
from jax import lax
from jax.experimental import pallas as pl
from jax.experimental.pallas import tpu as pltpu
from jax.experimental.pallas import tpu_sc as plsc

B = 16384
NF = 26
FIELD_SIZE = 100000
NC = 2
NS = 16
NW = NC * NS
ROWS_W = B // NW
FLAT_W = ROWS_W * NF
CH = 128
NCH = FLAT_W // CH
LANES = 16
DEPTH = 8

_OFF_PATTERN = (np.arange(FLAT_W, dtype=np.int64) % NF * FIELD_SIZE).astype(np.int32)


def _gather_sc(x3, offs, tab):
    mesh = plsc.VectorSubcoreMesh(
        core_axis_name="c", subcore_axis_name="s", num_cores=NC, num_subcores=NS
    )

    @functools.partial(
        pl.kernel,
        out_type=jax.ShapeDtypeStruct((NW * FLAT_W,), jnp.float32),
        mesh=mesh,
        scratch_types=[
            pltpu.VMEM((NCH, CH), jnp.int32),
            pltpu.VMEM((NCH, CH), jnp.int32),
            pltpu.VMEM((FLAT_W,), jnp.float32),
            pltpu.SemaphoreType.DMA,
        ],
    )
    def k(x_hbm, off_hbm, tab_hbm, out_hbm, idx_v, off_v, rows_v, sem):
        wid = lax.axis_index("s") * NC + lax.axis_index("c")
        pltpu.sync_copy(x_hbm.at[wid], idx_v)
        pltpu.sync_copy(off_hbm, off_v)

        def add_offsets(j, carry):
            for c in range(CH // LANES):
                sl = pl.ds(c * LANES, LANES)
                idx_v[j, sl] = idx_v[j, sl] + off_v[j, sl]
            return carry

        lax.fori_loop(0, NCH, add_offsets, 0)

        def fire(j):
            pltpu.async_copy(
                tab_hbm.at[idx_v.at[j]], rows_v.at[pl.ds(j * CH, CH)], sem
            )

        def drain(j):
            pltpu.make_async_copy(
                tab_hbm.at[idx_v.at[j]], rows_v.at[pl.ds(j * CH, CH)], sem
            ).wait()

        for j in range(DEPTH):
            fire(j)

        def steady(j, carry):
            fire(j + DEPTH)
            drain(j)
            return carry

        lax.fori_loop(0, NCH - DEPTH, steady, 0)

        def tail(j, carry):
            drain(j)
            return carry

        lax.fori_loop(NCH - DEPTH, NCH, tail, 0)
        pltpu.sync_copy(rows_v, out_hbm.at[pl.ds(wid * FLAT_W, FLAT_W)])

    return k(x3, offs, tab)


def _tc_body(g_ref, xc_ref, w_ref, b_ref, o_ref):
    emb = jnp.sum(g_ref[...], axis=1, keepdims=True)
    cont = jnp.sum(xc_ref[...] * w_ref[...], axis=1, keepdims=True)
    o_ref[...] = emb + cont + b_ref[0, 0]


def _linear_tc(gathered, x_cont, w2, bias2):
    blk = 2048
    return pl.pallas_call(
        _tc_body,
        grid=(B // blk,),
        in_specs=[
            pl.BlockSpec((blk, NF), lambda i: (i, 0)),
            pl.BlockSpec((blk, 128), lambda i: (i, 0)),
            pl.BlockSpec((1, 128), lambda i: (0, 0)),
            pl.BlockSpec((1, 1), lambda i: (0, 0)),
        ],
        out_specs=pl.BlockSpec((blk, 1), lambda i: (i, 0)),
        out_shape=jax.ShapeDtypeStruct((B, 1), jnp.float32),
    )(gathered, x_cont, w2, bias2)


def kernel(x, x_cont, emb_x, table, w, bias):
    x3 = x.reshape(NW, NCH, CH)
    offs = jnp.asarray(_OFF_PATTERN).reshape(NCH, CH)
    tab = table.reshape(-1)
    gathered = _gather_sc(x3, offs, tab).reshape(B, NF)
    return _linear_tc(gathered, x_cont, w.reshape(1, 128), bias.reshape(1, 1))


# speedup vs baseline: 1.1889x; 1.0001x over previous
import functools

import numpy as np

import jax
import jax.numpy as jnp
from jax import lax
from jax.experimental import pallas as pl
from jax.experimental.pallas import tpu as pltpu
from jax.experimental.pallas import tpu_sc as plsc

B = 16384
NF = 26
FIELD_SIZE = 100000
NC = 2
NS = 16
NW = NC * NS
ROWS_W = B // NW
FLAT_W = ROWS_W * NF
CH = 128
NCH = FLAT_W // CH
LANES = 16
DEPTH = 8

_OFF_PATTERN = (np.arange(FLAT_W, dtype=np.int64) % NF * FIELD_SIZE).astype(np.int32)


def _gather_sc(x3, offs, tab):
    mesh = plsc.VectorSubcoreMesh(
        core_axis_name="c", subcore_axis_name="s", num_cores=NC, num_subcores=NS
    )

    @functools.partial(
        pl.kernel,
        out_type=jax.ShapeDtypeStruct((NW * FLAT_W,), jnp.float32),
        mesh=mesh,
        scratch_types=[
            pltpu.VMEM((NCH, CH), jnp.int32),
            pltpu.VMEM((NCH, CH), jnp.int32),
            pltpu.VMEM((FLAT_W,), jnp.float32),
            pltpu.SemaphoreType.DMA,
        ],
    )
    def k(x_hbm, off_hbm, tab_hbm, out_hbm, idx_v, off_v, rows_v, sem):
        wid = lax.axis_index("s") * NC + lax.axis_index("c")
        pltpu.sync_copy(x_hbm.at[wid], idx_v)
        pltpu.sync_copy(off_hbm, off_v)

        def add_offsets(j, carry):
            for c in range(CH // LANES):
                sl = pl.ds(c * LANES, LANES)
                idx_v[j, sl] = idx_v[j, sl] + off_v[j, sl]
            return carry

        lax.fori_loop(0, NCH, add_offsets, 0)

        def fire(j):
            pltpu.async_copy(
                tab_hbm.at[idx_v.at[j]], rows_v.at[pl.ds(j * CH, CH)], sem
            )

        def drain(j):
            pltpu.make_async_copy(
                tab_hbm.at[idx_v.at[j]], rows_v.at[pl.ds(j * CH, CH)], sem
            ).wait()

        for j in range(DEPTH):
            fire(j)

        def steady(j, carry):
            fire(j + DEPTH)
            drain(j)
            return carry

        lax.fori_loop(0, NCH - DEPTH, steady, 0)

        def tail(j, carry):
            drain(j)
            return carry

        lax.fori_loop(NCH - DEPTH, NCH, tail, 0)
        pltpu.sync_copy(rows_v, out_hbm.at[pl.ds(wid * FLAT_W, FLAT_W)])

    return k(x3, offs, tab)


def _tc_body(g_ref, xc_ref, w_ref, b_ref, o_ref):
    emb = jnp.sum(g_ref[...], axis=1, keepdims=True)
    cont = jnp.sum(xc_ref[...] * w_ref[...], axis=1, keepdims=True)
    o_ref[...] = emb + cont + b_ref[0, 0]


def _linear_tc(gathered, x_cont, w2, bias2):
    blk = 2048
    return pl.pallas_call(
        _tc_body,
        grid=(B // blk,),
        in_specs=[
            pl.BlockSpec((blk, NF), lambda i: (i, 0)),
            pl.BlockSpec((blk, 128), lambda i: (i, 0)),
            pl.BlockSpec((1, 128), lambda i: (0, 0)),
            pl.BlockSpec((1, 1), lambda i: (0, 0)),
        ],
        out_specs=pl.BlockSpec((blk, 1), lambda i: (i, 0)),
        out_shape=jax.ShapeDtypeStruct((B, 1), jnp.float32),
    )(gathered, x_cont, w2, bias2)


def kernel(x, x_cont, emb_x, table, w, bias):
    x3 = x.reshape(NW, NCH, CH)
    offs = jnp.asarray(_OFF_PATTERN).reshape(NCH, CH)
    tab = table.T.reshape(-1)
    gathered = _gather_sc(x3, offs, tab).reshape(B, NF)
    return _linear_tc(gathered, x_cont, w.reshape(1, 128), bias.reshape(1, 1))


# speedup vs baseline: 1.4410x; 1.2121x over previous
import functools

import jax
import jax.numpy as jnp
from jax import lax
from jax.experimental import pallas as pl
from jax.experimental.pallas import tpu as pltpu
from jax.experimental.pallas import tpu_sc as plsc

B = 16384
NF = 26
FIELD_SIZE = 100000
NC = 2
NS = 16
NW = NC * NS
ROWS_W = B // NW
FLAT_W = ROWS_W * NF
CH = 128
CPF = ROWS_W // CH
NCH = FLAT_W // CH
LANES = 16
DEPTH = 8


def _emb_sum_sc(xt, tab):
    mesh = plsc.VectorSubcoreMesh(
        core_axis_name="c", subcore_axis_name="s", num_cores=NC, num_subcores=NS
    )

    @functools.partial(
        pl.kernel,
        out_type=jax.ShapeDtypeStruct((B,), jnp.float32),
        mesh=mesh,
        scratch_types=[
            pltpu.VMEM((NF, ROWS_W), jnp.int32),
            pltpu.VMEM((FLAT_W,), jnp.float32),
            pltpu.VMEM((ROWS_W,), jnp.float32),
            pltpu.SemaphoreType.DMA,
        ],
    )
    def k(xt_hbm, tab_hbm, out_hbm, idx_v, rows_v, acc_v, sem):
        wid = lax.axis_index("s") * NC + lax.axis_index("c")
        base = wid * ROWS_W
        pltpu.sync_copy(xt_hbm.at[:, pl.ds(base, ROWS_W)], idx_v)

        def add_offsets(f, carry):
            off = f * FIELD_SIZE
            for c in range(ROWS_W // LANES):
                sl = pl.ds(c * LANES, LANES)
                idx_v[f, sl] = idx_v[f, sl] + off
            return carry

        lax.fori_loop(0, NF, add_offsets, 0)

        def src(j):
            return tab_hbm.at[idx_v.at[j // CPF, pl.ds((j % CPF) * CH, CH)]]

        def fire(j):
            pltpu.async_copy(src(j), rows_v.at[pl.ds(j * CH, CH)], sem)

        def drain(j):
            pltpu.make_async_copy(src(j), rows_v.at[pl.ds(j * CH, CH)], sem).wait()

        for j in range(DEPTH):
            fire(j)

        def steady(j, carry):
            fire(j + DEPTH)
            drain(j)
            return carry

        lax.fori_loop(0, NCH - DEPTH, steady, 0)

        def tail(j, carry):
            drain(j)
            return carry

        lax.fori_loop(NCH - DEPTH, NCH, tail, 0)

        def row_sum(g, carry):
            acc = rows_v[pl.ds(g * LANES, LANES)]
            for f in range(1, NF):
                acc = acc + rows_v[pl.ds(f * ROWS_W + g * LANES, LANES)]
            acc_v[pl.ds(g * LANES, LANES)] = acc
            return carry

        lax.fori_loop(0, ROWS_W // LANES, row_sum, 0)
        pltpu.sync_copy(acc_v, out_hbm.at[pl.ds(base, ROWS_W)])

    return k(xt, tab)


def _tc_body(xc_ref, w_ref, b_ref, emb_ref, o_ref):
    cont = jnp.sum(xc_ref[...] * w_ref[...], axis=1)
    o_ref[...] = cont + emb_ref[...] + b_ref[0, 0]


def _linear_tc(x_cont, w2, bias2, emb):
    blk = 2048
    return pl.pallas_call(
        _tc_body,
        grid=(B // blk,),
        in_specs=[
            pl.BlockSpec((blk, 128), lambda i: (i, 0)),
            pl.BlockSpec((1, 128), lambda i: (0, 0)),
            pl.BlockSpec((1, 1), lambda i: (0, 0)),
            pl.BlockSpec((blk,), lambda i: (i,)),
        ],
        out_specs=pl.BlockSpec((blk,), lambda i: (i,)),
        out_shape=jax.ShapeDtypeStruct((B,), jnp.float32),
    )(x_cont, w2, bias2, emb)


def kernel(x, x_cont, emb_x, table, w, bias):
    xt = x.T
    tab = table.reshape(-1)
    emb = _emb_sum_sc(xt, tab)
    out = _linear_tc(x_cont, w.reshape(1, 128), bias.reshape(1, 1), emb)
    return out.reshape(B, 1)
